# baseline (speedup 1.0000x reference)
import functools

import jax
import jax.numpy as jnp
from jax import lax
from jax.experimental import pallas as pl
from jax.experimental.pallas import tpu as pltpu
from jax.experimental.pallas import tpu_sc as plsc

N_ELEC = 128
EMBED_DIM = 256
NUM_CORES = 2
NUM_SUBCORES = 16
NUM_WORKERS = NUM_CORES * NUM_SUBCORES
ROWS_PER_WORKER = N_ELEC // NUM_WORKERS


@functools.partial(
    pl.kernel,
    mesh=plsc.VectorSubcoreMesh(core_axis_name="c", subcore_axis_name="s"),
    out_type=jax.ShapeDtypeStruct((N_ELEC, EMBED_DIM), jnp.float32),
    scratch_types=[
        pltpu.VMEM((ROWS_PER_WORKER,), jnp.int32),
        pltpu.VMEM((ROWS_PER_WORKER, EMBED_DIM), jnp.float32),
        pltpu.SemaphoreType.DMA,
    ],
)
def _gather_kernel(table_hbm, idx_hbm, out_hbm, idx_v, rows_v, sem):
    wid = lax.axis_index("s") * NUM_CORES + lax.axis_index("c")
    pltpu.sync_copy(idx_hbm.at[wid], idx_v)
    pltpu.async_copy(table_hbm.at[idx_v], rows_v, sem).wait()
    pltpu.sync_copy(rows_v, out_hbm.at[pl.ds(wid * ROWS_PER_WORKER, ROWS_PER_WORKER)])


def kernel(phys_conf, nucleus_embedding, table, elec_types):
    del phys_conf, nucleus_embedding
    idx = elec_types.reshape(NUM_WORKERS, ROWS_PER_WORKER)
    return _gather_kernel(table, idx)


# speedup vs baseline: 1.0956x; 1.0956x over previous
import functools

import jax
import jax.numpy as jnp
from jax import lax
from jax.experimental import pallas as pl
from jax.experimental.pallas import tpu as pltpu
from jax.experimental.pallas import tpu_sc as plsc

N_ELEC = 128
EMBED_DIM = 256
N_TYPES = 2
LANES = 16
NUM_CORES = 2
NUM_SUBCORES = 16
NUM_WORKERS = NUM_CORES * NUM_SUBCORES
ROWS_PER_WORKER = N_ELEC // NUM_WORKERS
CHUNKS = EMBED_DIM // LANES


@functools.partial(
    pl.kernel,
    mesh=plsc.VectorSubcoreMesh(core_axis_name="c", subcore_axis_name="s"),
    out_type=jax.ShapeDtypeStruct((N_ELEC, EMBED_DIM), jnp.float32),
    scratch_types=[
        pltpu.VMEM((LANES,), jnp.int32),
        pltpu.VMEM((N_TYPES, EMBED_DIM), jnp.float32),
        pltpu.VMEM((ROWS_PER_WORKER, EMBED_DIM), jnp.float32),
        pltpu.SemaphoreType.DMA,
        pltpu.SemaphoreType.DMA,
    ],
)
def _embed_kernel(table_hbm, types_hbm, out_hbm, types_v, table_v, rows_v,
                  sem_a, sem_b):
    wid = lax.axis_index("s") * NUM_CORES + lax.axis_index("c")
    cp_types = pltpu.async_copy(types_hbm.at[wid], types_v, sem_a)
    cp_table = pltpu.async_copy(table_hbm, table_v, sem_b)
    cp_types.wait()
    cp_table.wait()
    tv = types_v[...].astype(jnp.float32)
    for c in range(CHUNKS):
        t0 = table_v[0, pl.ds(c * LANES, LANES)]
        diff = table_v[1, pl.ds(c * LANES, LANES)] - t0
        for r in range(ROWS_PER_WORKER):
            rows_v[r, pl.ds(c * LANES, LANES)] = t0 + tv[r] * diff
    pltpu.sync_copy(
        rows_v, out_hbm.at[pl.ds(wid * ROWS_PER_WORKER, ROWS_PER_WORKER)])


def kernel(phys_conf, nucleus_embedding, table, elec_types):
    del phys_conf, nucleus_embedding
    idx = elec_types.reshape(NUM_WORKERS, ROWS_PER_WORKER)
    idx = jnp.pad(idx, ((0, 0), (0, LANES - ROWS_PER_WORKER)))
    return _embed_kernel(table, idx)


# speedup vs baseline: 1.1981x; 1.0936x over previous
import functools

import jax
import jax.numpy as jnp
from jax import lax
from jax.experimental import pallas as pl
from jax.experimental.pallas import tpu as pltpu
from jax.experimental.pallas import tpu_sc as plsc

N_ELEC = 128
EMBED_DIM = 256
N_TYPES = 2
LANES = 16
NUM_CORES = 1
NUM_SUBCORES = 16
NUM_WORKERS = NUM_CORES * NUM_SUBCORES
ROWS_PER_WORKER = N_ELEC // NUM_WORKERS
CHUNKS = EMBED_DIM // LANES


@functools.partial(
    pl.kernel,
    mesh=plsc.VectorSubcoreMesh(
        core_axis_name="c", subcore_axis_name="s", num_cores=NUM_CORES),
    out_type=jax.ShapeDtypeStruct((N_ELEC, EMBED_DIM), jnp.float32),
    scratch_types=[
        pltpu.VMEM((LANES,), jnp.int32),
        pltpu.VMEM((N_TYPES, EMBED_DIM), jnp.float32),
        pltpu.VMEM((ROWS_PER_WORKER, EMBED_DIM), jnp.float32),
        pltpu.SemaphoreType.DMA,
        pltpu.SemaphoreType.DMA,
    ],
)
def _embed_kernel(table_hbm, types_hbm, out_hbm, types_v, table_v, rows_v,
                  sem_a, sem_b):
    wid = lax.axis_index("s") * NUM_CORES + lax.axis_index("c")
    cp_types = pltpu.async_copy(types_hbm.at[wid], types_v, sem_a)
    cp_table = pltpu.async_copy(table_hbm, table_v, sem_b)
    cp_types.wait()
    cp_table.wait()
    tv = types_v[...].astype(jnp.float32)
    for c in range(CHUNKS):
        t0 = table_v[0, pl.ds(c * LANES, LANES)]
        diff = table_v[1, pl.ds(c * LANES, LANES)] - t0
        for r in range(ROWS_PER_WORKER):
            rows_v[r, pl.ds(c * LANES, LANES)] = t0 + tv[r] * diff
    pltpu.sync_copy(
        rows_v, out_hbm.at[pl.ds(wid * ROWS_PER_WORKER, ROWS_PER_WORKER)])


def kernel(phys_conf, nucleus_embedding, table, elec_types):
    del phys_conf, nucleus_embedding
    idx = elec_types.reshape(NUM_WORKERS, ROWS_PER_WORKER)
    idx = jnp.pad(idx, ((0, 0), (0, LANES - ROWS_PER_WORKER)))
    return _embed_kernel(table, idx)
